# Initial kernel scaffold; baseline (speedup 1.0000x reference)
import jax
import jax.numpy as jnp
from jax.experimental import pallas as pl


def kernel(x_kv, x_q, spa_w, kv_w, kv_b, q_w, q_b, proj_w, proj_b, dw1_w, dw1_b, dw2_w, dw2_b):
    raise NotImplementedError("write your pallas kernel here")


# speedup vs baseline: 5.2616x; 5.2616x over previous
import jax
import jax.numpy as jnp
import numpy as np
from jax.experimental import pallas as pl
from jax.experimental.pallas import tpu as pltpu

_C = 256
_NH = 32
_HD = 8
_KK = 512
_D, _H, _W = 8, 16, 16
_N = _D * _H * _W
_HI = jax.lax.Precision.HIGHEST
_DF = jax.lax.Precision.DEFAULT


def _body_a(xkv_ref, xq_ref, band_ref, kvw_ref, kvb_ref, qw_ref, qb_ref,
            ksel_ref, vsel_ref, q_ref, v_ref):
    f32 = jnp.float32
    i32 = jnp.int32
    x = xkv_ref[0]
    xq = xq_ref[0]

    avg = jnp.mean(x, axis=1, keepdims=True)
    mx = jnp.max(x, axis=1, keepdims=True)
    bf = jnp.bfloat16
    avg3 = jnp.pad(avg.reshape(_D, _H, _W).astype(bf).astype(f32),
                   ((3, 3), (3, 3), (3, 3)))
    mx3 = jnp.pad(mx.reshape(_D, _H, _W).astype(bf).astype(f32),
                  ((3, 3), (3, 3), (3, 3)))
    acc = jnp.zeros((_D * _H, _W), f32)
    for c, vol in ((0, avg3), (1, mx3)):
        for dd in range(7):
            for hh in range(7):
                sl = vol[dd:dd + _D, hh:hh + _H, :].reshape(_D * _H, _W + 6)
                bnd = band_ref[c, dd, hh].astype(bf).astype(f32)
                acc = acc + jnp.dot(sl, bnd,
                                    preferred_element_type=f32, precision=_HI)

    raw = jax.lax.bitcast_convert_type(acc, i32)
    bits = jnp.where(raw < 0, jnp.bitwise_xor(raw, i32(0x7FFFFFFF)), raw)
    t = jnp.where(jnp.sum((bits >= 0).astype(i32)) >= _KK, i32(0),
                  i32(-2147483648))
    for b in range(30, -1, -1):
        cand = jnp.bitwise_or(t, i32(1 << b))
        cnt = jnp.sum((bits >= cand).astype(i32))
        t = jnp.where(cnt >= _KK, cand, t)
    gt = bits > t
    m = _KK - jnp.sum(gt.astype(i32))
    eq = bits == t
    n_arr = (16 * jax.lax.broadcasted_iota(i32, (_D * _H, _W), 0)
             + jax.lax.broadcasted_iota(i32, (_D * _H, _W), 1))
    j = i32(0)
    for b in range(11, -1, -1):
        cand = jnp.bitwise_or(j, i32(1 << b))
        cnt = jnp.sum((eq & (n_arr < cand)).astype(i32))
        j = jnp.where(cnt <= m, cand, j)
    mask_f = (gt | (eq & (n_arr < j))).astype(f32)

    ut16 = (jax.lax.broadcasted_iota(i32, (16, 16), 0)
            <= jax.lax.broadcasted_iota(i32, (16, 16), 1)).astype(f32)
    incl = jnp.dot(mask_f, ut16, preferred_element_type=f32, precision=_HI)
    slt = (jax.lax.broadcasted_iota(i32, (128, 128), 1)
           < jax.lax.broadcasted_iota(i32, (128, 128), 0)).astype(f32)
    offs = jnp.dot(slt, incl[:, 15:16], preferred_element_type=f32,
                   precision=_HI)
    rank = incl - mask_f + offs

    ri = jax.lax.broadcasted_iota(i32, (_N, 128), 0)
    ci16 = 16 * jax.lax.broadcasted_iota(i32, (_N, 128), 1)
    rm_col = jnp.zeros((_N, 2), f32)
    for jcol in range(16):
        pj = (ri == ci16 + jcol).astype(f32)
        rm_j = jnp.concatenate([rank[:, jcol:jcol + 1],
                                mask_f[:, jcol:jcol + 1]], axis=1)
        rm_col = rm_col + jnp.dot(pj, rm_j, preferred_element_type=f32,
                                  precision=_HI)
    rank_col = rm_col[:, 0:1].astype(i32)
    mask_col = rm_col[:, 1:2]

    k = jnp.dot(x, kvw_ref[:, :_C], preferred_element_type=f32,
                precision=_DF) + kvb_ref[:, :_C]
    v = jnp.dot(x, kvw_ref[:, _C:], preferred_element_type=f32,
                precision=_DF) + kvb_ref[:, _C:]
    tn = (((0,), (0,)), ((), ()))
    half = _KK // 2
    for i, lo in enumerate((0, half)):
        riota1 = lo + jax.lax.broadcasted_iota(i32, (_N, half), 1)
        gt_sel = jnp.where((rank_col == riota1) & (mask_col > 0.5), 1.0, 0.0)
        ksel_ref[0, i * half:(i + 1) * half, :] = jax.lax.dot_general(
            gt_sel, k, tn, preferred_element_type=f32, precision=_HI)
        vsel_ref[0, i * half:(i + 1) * half, :] = jax.lax.dot_general(
            gt_sel, v, tn, preferred_element_type=f32, precision=_HI)
    v_ref[0] = v
    q_ref[0] = jnp.dot(xq, qw_ref[...], preferred_element_type=f32,
                       precision=_DF) + qb_ref[...]


def _body_attn(qh_ref, kh_ref, vh_ref, oh_ref):
    f32 = jnp.float32
    q_h = qh_ref[0, 0]
    k_h = kh_ref[0, 0]
    v_h = vh_ref[0, 0]
    nt = (((1,), (1,)), ((), ()))
    s = jax.lax.dot_general(q_h, k_h, nt, preferred_element_type=f32,
                            precision=_DF) * (_HD ** -0.5)
    e = jnp.exp(s - jnp.max(s, axis=1, keepdims=True))
    o_h = jnp.dot(e, v_h, preferred_element_type=f32, precision=_DF)
    oh_ref[0, 0] = o_h / jnp.sum(e, axis=1, keepdims=True)


def _body_c(o_ref, v_ref, pw_ref, pb_ref,
            d1t_ref, d1b_ref, d2w_ref, d2b_ref, out_ref):
    f32 = jnp.float32
    v = v_ref[0]
    out_attn = jnp.dot(o_ref[0], pw_ref[...], preferred_element_type=f32,
                       precision=_DF) + pb_ref[...]

    v_p = jnp.pad(v.reshape(_D, _H, _W, _C),
                  ((1, 1), (1, 1), (1, 1), (0, 0)))
    r1 = jnp.zeros((_D, _H, _W, _C), f32)
    for dd in range(3):
        for hh in range(3):
            for ww in range(3):
                tap = dd * 9 + hh * 3 + ww
                r1 = r1 + (v_p[dd:dd + _D, hh:hh + _H, ww:ww + _W, :]
                           * d1t_ref[tap:tap + 1, :])
    r1 = (r1 + d1b_ref[...]).reshape(_N, _C)
    r2 = jnp.dot(r1, d2w_ref[...], preferred_element_type=f32,
                 precision=_DF) + d2b_ref[...]

    out_ref[0] = out_attn + r2


def _build_band(spa_w):
    cc, dd, hh, ww, w = np.meshgrid(np.arange(2), np.arange(7), np.arange(7),
                                    np.arange(7), np.arange(_W), indexing='ij')
    band = jnp.zeros((2, 7, 7, _W + 6, _W), jnp.float32)
    return band.at[cc, dd, hh, ww + w, w].set(spa_w[0][cc, dd, hh, ww])


@jax.jit
def _run(xf, xqf, band, kvwT, kvb, qwT, qb, pwT, pb, d1t, d1b, d2wT, d2b):
    B = xf.shape[0]
    full = lambda a: pl.BlockSpec(a.shape, lambda b: (0,) * a.ndim)
    tok = pl.BlockSpec((1, _N, _C), lambda b: (b, 0, 0))
    sel = pl.BlockSpec((1, _KK, _C), lambda b: (b, 0, 0))
    params = pltpu.CompilerParams(dimension_semantics=("arbitrary",))
    params2 = pltpu.CompilerParams(
        dimension_semantics=("arbitrary", "arbitrary"))
    f32 = jnp.float32
    k_sel, v_sel, q, v = pl.pallas_call(
        _body_a,
        grid=(B,),
        in_specs=[tok, tok, full(band), full(kvwT), full(kvb), full(qwT),
                  full(qb)],
        out_specs=[sel, sel, tok, tok],
        out_shape=[jax.ShapeDtypeStruct((B, _KK, _C), f32),
                   jax.ShapeDtypeStruct((B, _KK, _C), f32),
                   jax.ShapeDtypeStruct((B, _N, _C), f32),
                   jax.ShapeDtypeStruct((B, _N, _C), f32)],
        compiler_params=params,
    )(xf, xqf, band, kvwT, kvb, qwT, qb)
    qh = q.reshape(B, _N, _NH, _HD).transpose(0, 2, 1, 3)
    kh = k_sel.reshape(B, _KK, _NH, _HD).transpose(0, 2, 1, 3)
    vh = v_sel.reshape(B, _KK, _NH, _HD).transpose(0, 2, 1, 3)
    hblk = lambda n: pl.BlockSpec((1, 1, n, _HD), lambda b, h: (b, h, 0, 0))
    oh = pl.pallas_call(
        _body_attn,
        grid=(B, _NH),
        in_specs=[hblk(_N), hblk(_KK), hblk(_KK)],
        out_specs=hblk(_N),
        out_shape=jax.ShapeDtypeStruct((B, _NH, _N, _HD), f32),
        compiler_params=params2,
    )(qh, kh, vh)
    o = oh.transpose(0, 2, 1, 3).reshape(B, _N, _C)
    return pl.pallas_call(
        _body_c,
        grid=(B,),
        in_specs=[tok, tok, full(pwT), full(pb), full(d1t),
                  full(d1b), full(d2wT), full(d2b)],
        out_specs=tok,
        out_shape=jax.ShapeDtypeStruct((B, _N, _C), f32),
        compiler_params=params,
    )(o, v, pwT, pb, d1t, d1b, d2wT, d2b)


def kernel(x_kv, x_q, spa_w, kv_w, kv_b, q_w, q_b, proj_w, proj_b,
           dw1_w, dw1_b, dw2_w, dw2_b):
    B, C, D, H, W = x_kv.shape
    xf = x_kv.reshape(B, C, _N).transpose(0, 2, 1)
    xqf = x_q.reshape(B, C, _N).transpose(0, 2, 1)
    out = _run(xf, xqf, _build_band(spa_w), kv_w.T, kv_b.reshape(1, -1),
               q_w.T, q_b.reshape(1, -1), proj_w.T, proj_b.reshape(1, -1),
               dw1_w.reshape(C, 27).T, dw1_b.reshape(1, -1),
               dw2_w.reshape(C, C).T, dw2_b.reshape(1, -1))
    return out.transpose(0, 2, 1).reshape(B, C, D, H, W)
